# Initial kernel scaffold; baseline (speedup 1.0000x reference)
import jax
import jax.numpy as jnp
from jax.experimental import pallas as pl


def kernel(feats, edge_index, W1, b1, W2, b2, dec1_W, dec1_b, dec2_W, dec2_b, lin_W, lin_b, codebook):
    raise NotImplementedError("write your pallas kernel here")


# speedup vs baseline: 1.0099x; 1.0099x over previous
import functools

import jax
import jax.numpy as jnp
from jax import lax
from jax.experimental import pallas as pl
from jax.experimental.pallas import tpu as pltpu

_N = 8192
_E = 131072
_D = 256
_H = 256
_O = 128
_K = 1024
_RB = 512
_SENT = 1 << 20


def _prep_body(dgo_ref, dgi_ref, feats_ref, ns_ref, nd_ref, hs_ref):
    dgo = dgo_ref[:, 0:1]
    dgi = dgi_ref[:, 0:1]
    ns = jnp.where(dgo > 0, lax.rsqrt(jnp.maximum(dgo, 1.0)), 0.0)
    nd = jnp.where(dgi > 0, lax.rsqrt(jnp.maximum(dgi, 1.0)), 0.0)
    ns_ref[...] = ns
    nd_ref[...] = nd
    hs_ref[...] = feats_ref[...] * ns


def _prep(deg_out, deg_in, feats):
    nb = _N // _RB
    return pl.pallas_call(
        _prep_body,
        grid=(nb,),
        in_specs=[
            pl.BlockSpec((_RB, 16), lambda i: (i, 0)),
            pl.BlockSpec((_RB, 16), lambda i: (i, 0)),
            pl.BlockSpec((_RB, _D), lambda i: (i, 0)),
        ],
        out_specs=[
            pl.BlockSpec((_RB, 1), lambda i: (i, 0)),
            pl.BlockSpec((_RB, 1), lambda i: (i, 0)),
            pl.BlockSpec((_RB, _D), lambda i: (i, 0)),
        ],
        out_shape=[
            jax.ShapeDtypeStruct((_N, 1), jnp.float32),
            jax.ShapeDtypeStruct((_N, 1), jnp.float32),
            jax.ShapeDtypeStruct((_N, _D), jnp.float32),
        ],
    )(deg_out, deg_in, feats)


def _pc1_body(agg_ref, nd_ref, W_ref, b_ref, h_ref, xn_ref):
    a = agg_ref[...] * nd_ref[...]
    h = jnp.maximum(
        jnp.dot(a, W_ref[...], preferred_element_type=jnp.float32) + b_ref[...], 0.0)
    h_ref[...] = h
    nrm = jnp.sqrt(jnp.sum(h * h, axis=1, keepdims=True))
    xn_ref[...] = h / (nrm + 1e-12)


def _postconv1(agg, nd, W1, b1):
    nb = _N // _RB
    return pl.pallas_call(
        _pc1_body,
        grid=(nb,),
        in_specs=[
            pl.BlockSpec((_RB, _D), lambda i: (i, 0)),
            pl.BlockSpec((_RB, 1), lambda i: (i, 0)),
            pl.BlockSpec((_D, _D), lambda i: (0, 0)),
            pl.BlockSpec((1, _D), lambda i: (0, 0)),
        ],
        out_specs=[
            pl.BlockSpec((_RB, _D), lambda i: (i, 0)),
            pl.BlockSpec((_RB, _D), lambda i: (i, 0)),
        ],
        out_shape=[
            jax.ShapeDtypeStruct((_N, _D), jnp.float32),
            jax.ShapeDtypeStruct((_N, _D), jnp.float32),
        ],
    )(agg, nd, W1, b1)


def _cbn_body(cb_ref, out_ref):
    cb = cb_ref[...]
    nrm = jnp.sqrt(jnp.sum(cb * cb, axis=1, keepdims=True))
    out_ref[...] = cb / (nrm + 1e-12)


def _cbnorm(cb):
    return pl.pallas_call(
        _cbn_body,
        out_shape=jax.ShapeDtypeStruct((_K, _D), jnp.float32),
    )(cb)


def _vq_body(xn_ref, h_ref, cbn_ref, dist_ref, quant_ref, commit_ref):
    i = pl.program_id(0)
    xn = xn_ref[...]
    cbn = cbn_ref[...]
    dist = lax.dot_general(xn, cbn, (((1,), (1,)), ((), ())),
                           preferred_element_type=jnp.float32)
    dist_ref[...] = dist
    mx = jnp.max(dist, axis=1, keepdims=True)
    kio = lax.broadcasted_iota(jnp.int32, dist.shape, 1)
    ind = jnp.min(jnp.where(dist >= mx, kio, _K), axis=1, keepdims=True)
    onehot = (kio == ind).astype(jnp.float32)
    quant = jnp.dot(onehot, cbn, preferred_element_type=jnp.float32,
                    precision=lax.Precision.HIGHEST)
    quant_ref[...] = quant
    d = quant - h_ref[...]

    @pl.when(i == 0)
    def _():
        commit_ref[...] = jnp.zeros_like(commit_ref)

    commit_ref[...] += jnp.sum(d * d)


def _vq(xn, h, cbn):
    nb = _N // _RB
    return pl.pallas_call(
        _vq_body,
        grid=(nb,),
        in_specs=[
            pl.BlockSpec((_RB, _D), lambda i: (i, 0)),
            pl.BlockSpec((_RB, _D), lambda i: (i, 0)),
            pl.BlockSpec((_K, _D), lambda i: (0, 0)),
        ],
        out_specs=[
            pl.BlockSpec((_RB, _K), lambda i: (i, 0)),
            pl.BlockSpec((_RB, _D), lambda i: (i, 0)),
            pl.BlockSpec((1, 1), lambda i: (0, 0)),
        ],
        out_shape=[
            jax.ShapeDtypeStruct((_N, _K), jnp.float32),
            jax.ShapeDtypeStruct((_N, _D), jnp.float32),
            jax.ShapeDtypeStruct((1, 1), jnp.float32),
        ],
    )(xn, h, cbn)


def _dec_body(qt_ref, h_ref, ns_ref, d1w_ref, d1b_ref, d2w_ref, d2b_ref,
              qe_ref, qs_ref, feat_ref, srow_ref, M_ref):
    i = pl.program_id(0)
    qt = qt_ref[...]
    qe = jnp.dot(qt, d1w_ref[...], preferred_element_type=jnp.float32) + d1b_ref[...]
    qn = jnp.dot(qt, d2w_ref[...], preferred_element_type=jnp.float32) + d2b_ref[...]
    qe_ref[...] = qe
    qs_ref[...] = qe * ns_ref[...]
    d = h_ref[...] - qn

    @pl.when(i == 0)
    def _():
        feat_ref[...] = jnp.zeros_like(feat_ref)
        srow_ref[...] = jnp.zeros_like(srow_ref)
        M_ref[...] = jnp.zeros_like(M_ref)

    feat_ref[...] += jnp.sum(d * d)
    srow_ref[...] += jnp.sum(qe, axis=0, keepdims=True)
    M_ref[...] += lax.dot_general(qe, qe, (((0,), (0,)), ((), ())),
                                  preferred_element_type=jnp.float32)


def _decode(quant, h, ns, d1w, d1b, d2w, d2b):
    nb = _N // _RB
    return pl.pallas_call(
        _dec_body,
        grid=(nb,),
        in_specs=[
            pl.BlockSpec((_RB, _D), lambda i: (i, 0)),
            pl.BlockSpec((_RB, _D), lambda i: (i, 0)),
            pl.BlockSpec((_RB, 1), lambda i: (i, 0)),
            pl.BlockSpec((_D, _D), lambda i: (0, 0)),
            pl.BlockSpec((1, _D), lambda i: (0, 0)),
            pl.BlockSpec((_D, _D), lambda i: (0, 0)),
            pl.BlockSpec((1, _D), lambda i: (0, 0)),
        ],
        out_specs=[
            pl.BlockSpec((_RB, _D), lambda i: (i, 0)),
            pl.BlockSpec((_RB, _D), lambda i: (i, 0)),
            pl.BlockSpec((1, 1), lambda i: (0, 0)),
            pl.BlockSpec((1, _D), lambda i: (0, 0)),
            pl.BlockSpec((_D, _D), lambda i: (0, 0)),
        ],
        out_shape=[
            jax.ShapeDtypeStruct((_N, _D), jnp.float32),
            jax.ShapeDtypeStruct((_N, _D), jnp.float32),
            jax.ShapeDtypeStruct((1, 1), jnp.float32),
            jax.ShapeDtypeStruct((1, _D), jnp.float32),
            jax.ShapeDtypeStruct((_D, _D), jnp.float32),
        ],
    )(quant, h, ns, d1w, d1b, d2w, d2b)


def _mm_body(a_ref, b_ref, mn_ref, mx_ref):
    i = pl.program_id(0)
    j = pl.program_id(1)
    q = lax.dot_general(a_ref[...], b_ref[...], (((1,), (1,)), ((), ())),
                        preferred_element_type=jnp.float32)

    @pl.when((i == 0) & (j == 0))
    def _():
        mn_ref[...] = jnp.full(mn_ref.shape, jnp.inf, jnp.float32)
        mx_ref[...] = jnp.full(mx_ref.shape, -jnp.inf, jnp.float32)

    mn_ref[...] = jnp.minimum(mn_ref[...], jnp.min(q))
    mx_ref[...] = jnp.maximum(mx_ref[...], jnp.max(q))


def _minmax(qe):
    nb = _N // _RB
    return pl.pallas_call(
        _mm_body,
        grid=(nb, nb),
        in_specs=[
            pl.BlockSpec((_RB, _D), lambda i, j: (i, 0)),
            pl.BlockSpec((_RB, _D), lambda i, j: (j, 0)),
        ],
        out_specs=[
            pl.BlockSpec((1, 1), lambda i, j: (0, 0)),
            pl.BlockSpec((1, 1), lambda i, j: (0, 0)),
        ],
        out_shape=[
            jax.ShapeDtypeStruct((1, 1), jnp.float32),
            jax.ShapeDtypeStruct((1, 1), jnp.float32),
        ],
    )(qe, qe)


def _keys_body(ei_ref, k_ref):
    k_ref[...] = ei_ref[0:1, :] * _N + ei_ref[1:2, :]


def _keys(edge_index):
    return pl.pallas_call(
        _keys_body,
        out_shape=jax.ShapeDtypeStruct((1, _E), jnp.int32),
    )(edge_index)


def _mask_body(sk_ref, pv_ref, srcs_ref, dste_ref, u_ref):
    k = sk_ref[...]
    m = k != pv_ref[...]
    srcs_ref[...] = lax.shift_right_logical(k, 13)
    dst = jnp.bitwise_and(k, _N - 1)
    dste_ref[...] = jnp.where(m, dst, _SENT)
    u_ref[...] = jnp.sum(m.astype(jnp.float32)).reshape(1, 1)


def _mask(sk2, pv2):
    return pl.pallas_call(
        _mask_body,
        out_shape=[
            jax.ShapeDtypeStruct((_E // 128, 128), jnp.int32),
            jax.ShapeDtypeStruct((_E // 128, 128), jnp.int32),
            jax.ShapeDtypeStruct((1, 1), jnp.float32),
        ],
    )(sk2, pv2)


def _pc2_body(agg_ref, nd_ref, W_ref, b_ref, lw_ref, lb_ref, pu_ref, qe_ref,
              out_ref, t_ref):
    i = pl.program_id(0)
    a = agg_ref[...] * nd_ref[...]
    h2 = jnp.maximum(
        jnp.dot(a, W_ref[...], preferred_element_type=jnp.float32) + b_ref[...], 0.0)
    out_ref[...] = jnp.dot(h2, lw_ref[...],
                           preferred_element_type=jnp.float32) + lb_ref[...]

    @pl.when(i == 0)
    def _():
        t_ref[...] = jnp.zeros_like(t_ref)

    t_ref[...] += jnp.sum(pu_ref[...] * qe_ref[...])


def _postconv2(agg2, nd, W2, b2, lw, lb, pu, qe):
    nb = _N // _RB
    return pl.pallas_call(
        _pc2_body,
        grid=(nb,),
        in_specs=[
            pl.BlockSpec((_RB, _H), lambda i: (i, 0)),
            pl.BlockSpec((_RB, 1), lambda i: (i, 0)),
            pl.BlockSpec((_H, _H), lambda i: (0, 0)),
            pl.BlockSpec((1, _H), lambda i: (0, 0)),
            pl.BlockSpec((_H, _O), lambda i: (0, 0)),
            pl.BlockSpec((1, _O), lambda i: (0, 0)),
            pl.BlockSpec((_RB, _D), lambda i: (i, 0)),
            pl.BlockSpec((_RB, _D), lambda i: (i, 0)),
        ],
        out_specs=[
            pl.BlockSpec((_RB, _O), lambda i: (i, 0)),
            pl.BlockSpec((1, 1), lambda i: (0, 0)),
        ],
        out_shape=[
            jax.ShapeDtypeStruct((_N, _O), jnp.float32),
            jax.ShapeDtypeStruct((1, 1), jnp.float32),
        ],
    )(agg2, nd, W2, b2, lw, lb, pu, qe)


def _fin_body(cs_ref, fs_ref, M_ref, s_ref, mn_ref, mx_ref, t_ref, u_ref,
              loss_ref, fr_ref, er_ref, cl_ref):
    M = M_ref[...]
    sq2 = jnp.sum(M * M)
    sr = s_ref[...]
    sq = jnp.sum(sr * sr)
    mn = mn_ref[0, 0]
    mx = mx_ref[0, 0]
    T = t_ref[0, 0]
    U = u_ref[0, 0]
    den = mx - mn
    n2 = jnp.float32(_N) * jnp.float32(_N)
    nd_el = jnp.float32(_N) * jnp.float32(_D)
    sq2n = (sq2 - 2.0 * mn * sq + n2 * mn * mn) / (den * den)
    tn = (T - U * mn) / den
    S = sq2n - 2.0 * tn + U
    er = 0.3 * jnp.sqrt(S / n2)
    fr = 0.7 * fs_ref[0, 0] / nd_el
    cl = 0.25 * cs_ref[0, 0] / nd_el
    fr_ref[...] = jnp.full((1, 1), fr)
    er_ref[...] = jnp.full((1, 1), er)
    cl_ref[...] = jnp.full((1, 1), cl)
    loss_ref[...] = jnp.full((1, 1), fr + er + cl)


def _finalize(cs, fs, M, srow, mn, mx, T, U):
    return pl.pallas_call(
        _fin_body,
        out_shape=[jax.ShapeDtypeStruct((1, 1), jnp.float32)] * 4,
    )(cs, fs, M, srow, mn, mx, T, U)


def _scat_degrees(src, dst):
    dgo = jnp.zeros((_N,), jnp.float32).at[src].add(1.0)
    dgi = jnp.zeros((_N,), jnp.float32).at[dst].add(1.0)
    return (jnp.broadcast_to(dgo[:, None], (_N, 16)),
            jnp.broadcast_to(dgi[:, None], (_N, 16)))


def _scat_rows(table, gidx, sidx):
    return jnp.zeros((_N, table.shape[1]), jnp.float32).at[sidx].add(
        table[gidx], mode="drop")


def kernel(feats, edge_index, W1, b1, W2, b2, dec1_W, dec1_b, dec2_W, dec2_b,
           lin_W, lin_b, codebook):
    src = edge_index[0]
    dst = edge_index[1]
    b1r = b1.reshape(1, _D)
    b2r = b2.reshape(1, _H)
    d1br = dec1_b.reshape(1, _D)
    d2br = dec2_b.reshape(1, _D)
    lbr = lin_b.reshape(1, _O)

    deg_out, deg_in = _scat_degrees(src, dst)
    ns, nd, h_scaled = _prep(deg_out, deg_in, feats)

    agg1 = _scat_rows(h_scaled, src, dst)
    h, xn = _postconv1(agg1, nd, W1, b1r)
    cbn = _cbnorm(codebook)
    dist, quant, commit_s = _vq(xn, h, cbn)
    q_edge, q_s, feat_s, srow, M = _decode(quant, h, ns, dec1_W, d1br,
                                           dec2_W, d2br)

    keys = _keys(edge_index).reshape(_E)
    sk = jnp.sort(keys)
    pv = jnp.concatenate([jnp.full((1,), -1, jnp.int32), sk[:-1]])
    srcs2, dste2, U = _mask(sk.reshape(_E // 128, 128),
                            pv.reshape(_E // 128, 128))
    src_u = srcs2.reshape(_E)
    dst_u = dste2.reshape(_E)

    mn, mx = _minmax(q_edge)
    agg2 = _scat_rows(q_s, src, dst)
    p_u = _scat_rows(q_edge, src_u, dst_u)
    out, T = _postconv2(agg2, nd, W2, b2r, lin_W, lbr, p_u, q_edge)
    loss, fr, er, cl = _finalize(commit_s, feat_s, M, srow, mn, mx, T, U)

    return (out, loss.reshape(()), dist, cbn, fr.reshape(()),
            er.reshape(()), cl.reshape(()))
